# baseline (speedup 1.0000x reference)
import jax
import jax.numpy as jnp
from jax import lax
from jax.experimental import pallas as pl
from jax.experimental.pallas import tpu as pltpu
from jax.experimental.pallas import tpu_sc as plsc

VOCAB = 1000000
D = 64
PID_BASE = VOCAB - 3
NC, NS, L = 2, 16, 16
NW = NC * NS

BATCH = 4096
SEQ = 200
BL = 128
BT = BATCH // BL
LT, LI = SEQ // 8, 8
DT, DI = D // 8, 8


def _sc_body(idx_hbm, pp_hbm, table_hbm, out_hbm,
             idx_v, rows_v, trans_v, pp_v, pos_v, off_v, sem0, sem1):
    wid = lax.axis_index("s") * NC + lax.axis_index("c")
    sems = (sem0, sem1)

    pltpu.sync_copy(pp_hbm, pp_v)
    for lt in range(LT):
        pltpu.sync_copy(idx_hbm.at[lt, wid], idx_v.at[lt])

    lanes = lax.iota(jnp.int32, L)

    def fire(blk, buf):
        pltpu.async_copy(table_hbm.at[idx_v.at[blk // LI, blk % LI]],
                         rows_v.at[buf], sems[buf])

    def finish(blk, buf):
        pltpu.make_async_copy(table_hbm.at[idx_v.at[blk // LI, blk % LI]],
                              rows_v.at[buf], sems[buf]).wait()

        cnt = jnp.int32(0)
        for k in range(BL // L):
            v = idx_v[blk // LI, blk % LI, pl.ds(k * L, L)]
            m = v >= PID_BASE
            loc = lanes + k * L
            plsc.store_compressed(pos_v.at[pl.ds(cnt, L)], loc, mask=m)
            plsc.store_compressed(off_v.at[pl.ds(cnt, L)], v - PID_BASE,
                                  mask=m)
            cnt = cnt + jnp.sum(m.astype(jnp.int32))

        def fix(i, c):
            p = pos_v[pl.ds(i, L)][0]
            o = off_v[pl.ds(i, L)][0]
            for kk in range(D // L):
                sl = pl.ds(kk * L, L)
                rows_v[buf, p, sl] = pp_v[o, sl]
            return c

        lax.fori_loop(0, cnt, fix, 0)

        for dt in range(DT):
            for di in range(DI):
                d = dt * DI + di
                cvec = jnp.full((L,), d, jnp.int32)
                for g in range(BL // L):
                    rvec = lanes + g * L
                    vals = plsc.load_gather(rows_v.at[buf], [rvec, cvec])
                    trans_v[dt, di, pl.ds(g * L, L)] = vals

        pltpu.sync_copy(trans_v, out_hbm.at[blk, :, wid])

    fire(0, 0)

    def loop_body(t, carry):
        blk0 = 2 * t
        fire(blk0 + 1, 1)
        finish(blk0, 0)

        @pl.when(blk0 + 2 < SEQ)
        def _():
            fire(blk0 + 2, 0)

        finish(blk0 + 1, 1)
        return carry

    lax.fori_loop(0, SEQ // 2, loop_body, 0)


@jax.jit
def _run(idx5, prompt_params, table):
    mesh = plsc.VectorSubcoreMesh(core_axis_name="c", subcore_axis_name="s",
                                  num_cores=NC, num_subcores=NS)
    f = pl.kernel(
        _sc_body,
        out_type=jax.ShapeDtypeStruct((SEQ, DT, BT, DI, BL), jnp.float32),
        mesh=mesh,
        scratch_types=[
            pltpu.VMEM((LT, LI, BL), jnp.int32),
            pltpu.VMEM((2, BL, D), jnp.float32),
            pltpu.VMEM((DT, DI, BL), jnp.float32),
            pltpu.VMEM((4, D), jnp.float32),
            pltpu.VMEM((BL + L,), jnp.int32),
            pltpu.VMEM((BL + L,), jnp.int32),
            pltpu.SemaphoreType.DMA,
            pltpu.SemaphoreType.DMA,
        ],
        compiler_params=pltpu.CompilerParams(needs_layout_passes=False,
                                             use_tc_tiling_on_sc=False),
    )
    return f(idx5, prompt_params, table)


def kernel(input, table, prompt_params):
    idx = input.astype(jnp.int32)
    idx5 = idx.T.reshape(LT, LI, BT, BL).transpose(0, 2, 1, 3)
    pp = jnp.concatenate(
        [prompt_params.astype(jnp.float32),
         jnp.zeros((1, D), jnp.float32)], axis=0)
    out5 = _run(idx5, pp, table)
    return out5.transpose(2, 4, 0, 1, 3).reshape(BATCH, SEQ, D)


# speedup vs baseline: 1.5936x; 1.5936x over previous
import jax
import jax.numpy as jnp
from jax import lax
from jax.experimental import pallas as pl
from jax.experimental.pallas import tpu as pltpu
from jax.experimental.pallas import tpu_sc as plsc

VOCAB = 1000000
D = 64
PID_BASE = VOCAB - 3
NC, NS, L = 2, 16, 16
NW = NC * NS

BATCH = 4096
SEQ = 200
BL = 128
BT = BATCH // BL
LT, LI = SEQ // 8, 8
DT, DI = D // 8, 8
TROW = 129


def _sc_body(idx_hbm, pp_hbm, table_hbm, out_hbm,
             idx_v, rows_v, trans_v, pp_v, pos_v, off_v, sem0, sem1):
    wid = lax.axis_index("s") * NC + lax.axis_index("c")
    sems = (sem0, sem1)

    pltpu.sync_copy(pp_hbm, pp_v)
    for lt in range(LT):
        pltpu.sync_copy(idx_hbm.at[lt, wid], idx_v.at[lt])

    lanes = lax.iota(jnp.int32, L)

    def fire(blk, buf):
        pltpu.async_copy(table_hbm.at[idx_v.at[blk // LI, blk % LI]],
                         rows_v.at[buf], sems[buf])

    def finish(blk, buf):
        pltpu.make_async_copy(table_hbm.at[idx_v.at[blk // LI, blk % LI]],
                              rows_v.at[buf], sems[buf]).wait()

        cnt = jnp.int32(0)
        for k in range(BL // L):
            v = idx_v[blk // LI, blk % LI, pl.ds(k * L, L)]
            m = v >= PID_BASE
            loc = lanes + k * L
            plsc.store_compressed(pos_v.at[pl.ds(cnt, L)], loc, mask=m)
            plsc.store_compressed(off_v.at[pl.ds(cnt, L)], v - PID_BASE,
                                  mask=m)
            cnt = cnt + jnp.sum(m.astype(jnp.int32))

        def fix(i, c):
            p = pos_v[pl.ds(i, L)][0]
            o = off_v[pl.ds(i, L)][0]
            for kk in range(D // L):
                sl = pl.ds(kk * L, L)
                rows_v[buf, p, sl] = pp_v[o, sl]
            return c

        lax.fori_loop(0, cnt, fix, 0)

        for r in range(BL):
            for g4 in range(D // L):
                dvec = lanes + g4 * L
                cvec = jnp.full((L,), r, jnp.int32)
                vals = rows_v[buf, r, pl.ds(g4 * L, L)]
                plsc.store_scatter(trans_v, [dvec, cvec], vals)

        for dt in range(DT):
            pltpu.sync_copy(trans_v.at[pl.ds(dt * DI, DI), pl.ds(0, BL)],
                            out_hbm.at[blk, dt, wid])

    fire(0, 0)

    def loop_body(t, carry):
        blk0 = 2 * t
        fire(blk0 + 1, 1)
        finish(blk0, 0)

        @pl.when(blk0 + 2 < SEQ)
        def _():
            fire(blk0 + 2, 0)

        finish(blk0 + 1, 1)
        return carry

    lax.fori_loop(0, SEQ // 2, loop_body, 0)


@jax.jit
def _run(idx5, prompt_params, table):
    mesh = plsc.VectorSubcoreMesh(core_axis_name="c", subcore_axis_name="s",
                                  num_cores=NC, num_subcores=NS)
    f = pl.kernel(
        _sc_body,
        out_type=jax.ShapeDtypeStruct((SEQ, DT, BT, DI, BL), jnp.float32),
        mesh=mesh,
        scratch_types=[
            pltpu.VMEM((LT, LI, BL), jnp.int32),
            pltpu.VMEM((2, BL, D), jnp.float32),
            pltpu.VMEM((D, TROW), jnp.float32),
            pltpu.VMEM((4, D), jnp.float32),
            pltpu.VMEM((BL + L,), jnp.int32),
            pltpu.VMEM((BL + L,), jnp.int32),
            pltpu.SemaphoreType.DMA,
            pltpu.SemaphoreType.DMA,
        ],
        compiler_params=pltpu.CompilerParams(needs_layout_passes=False,
                                             use_tc_tiling_on_sc=False),
    )
    return f(idx5, prompt_params, table)


def kernel(input, table, prompt_params):
    idx = input.astype(jnp.int32)
    idx5 = idx.T.reshape(LT, LI, BT, BL).transpose(0, 2, 1, 3)
    pp = jnp.concatenate(
        [prompt_params.astype(jnp.float32),
         jnp.zeros((1, D), jnp.float32)], axis=0)
    out5 = _run(idx5, pp, table)
    return out5.transpose(2, 4, 0, 1, 3).reshape(BATCH, SEQ, D)


# speedup vs baseline: 1.6949x; 1.0636x over previous
import jax
import jax.numpy as jnp
from jax import lax
from jax.experimental import pallas as pl
from jax.experimental.pallas import tpu as pltpu
from jax.experimental.pallas import tpu_sc as plsc

VOCAB = 1000000
D = 64
PID_BASE = VOCAB - 3
NC, NS, L = 2, 16, 16
NW = NC * NS

BATCH = 4096
SEQ = 200
BL = 128
BT = BATCH // BL
LT, LI = SEQ // 8, 8
DT, DI = D // 8, 8
TROW = 129


def _sc_body(idx_hbm, pp_hbm, table_hbm, out_hbm,
             idx_v, rows_v, trans_v, pp_v, pos_v, off_v,
             sem0, sem1, osem0, osem1):
    wid = lax.axis_index("s") * NC + lax.axis_index("c")
    sems = (sem0, sem1)
    osems = (osem0, osem1)

    pltpu.sync_copy(pp_hbm, pp_v)
    for lt in range(LT):
        pltpu.sync_copy(idx_hbm.at[lt, wid], idx_v.at[lt])

    lanes = lax.iota(jnp.int32, L)

    def fire(blk, buf):
        pltpu.async_copy(table_hbm.at[idx_v.at[blk // LI, blk % LI]],
                         rows_v.at[buf], sems[buf])

    def finish(blk, buf):
        pltpu.make_async_copy(table_hbm.at[idx_v.at[blk // LI, blk % LI]],
                              rows_v.at[buf], sems[buf]).wait()

        cnt = jnp.int32(0)
        for k in range(BL // L):
            v = idx_v[blk // LI, blk % LI, pl.ds(k * L, L)]
            m = v >= PID_BASE
            loc = lanes + k * L
            plsc.store_compressed(pos_v.at[pl.ds(cnt, L)], loc, mask=m)
            plsc.store_compressed(off_v.at[pl.ds(cnt, L)], v - PID_BASE,
                                  mask=m)
            cnt = cnt + jnp.sum(m.astype(jnp.int32))

        def fix(i, c):
            p = pos_v[pl.ds(i, L)][0]
            o = off_v[pl.ds(i, L)][0]
            for kk in range(D // L):
                sl = pl.ds(kk * L, L)
                rows_v[buf, p, sl] = pp_v[o, sl]
            return c

        lax.fori_loop(0, cnt, fix, 0)

        @pl.when(blk >= 2)
        def _():
            for dt in range(DT):
                pltpu.make_async_copy(
                    trans_v.at[buf, pl.ds(dt * DI, DI), pl.ds(0, BL)],
                    out_hbm.at[blk - 2, dt, wid], osems[buf]).wait()

        for r in range(BL):
            for g4 in range(D // L):
                dvec = lanes + g4 * L
                cvec = jnp.full((L,), r, jnp.int32)
                vals = rows_v[buf, r, pl.ds(g4 * L, L)]
                plsc.store_scatter(trans_v.at[buf], [dvec, cvec], vals)

        for dt in range(DT):
            pltpu.async_copy(trans_v.at[buf, pl.ds(dt * DI, DI), pl.ds(0, BL)],
                             out_hbm.at[blk, dt, wid], osems[buf])

    fire(0, 0)

    def loop_body(t, carry):
        blk0 = 2 * t
        fire(blk0 + 1, 1)
        finish(blk0, 0)

        @pl.when(blk0 + 2 < SEQ)
        def _():
            fire(blk0 + 2, 0)

        finish(blk0 + 1, 1)
        return carry

    lax.fori_loop(0, SEQ // 2, loop_body, 0)

    for blk, buf in ((SEQ - 2, 0), (SEQ - 1, 1)):
        for dt in range(DT):
            pltpu.make_async_copy(
                trans_v.at[buf, pl.ds(dt * DI, DI), pl.ds(0, BL)],
                out_hbm.at[blk, dt, wid], osems[buf]).wait()


@jax.jit
def _run(idx5, prompt_params, table):
    mesh = plsc.VectorSubcoreMesh(core_axis_name="c", subcore_axis_name="s",
                                  num_cores=NC, num_subcores=NS)
    f = pl.kernel(
        _sc_body,
        out_type=jax.ShapeDtypeStruct((SEQ, DT, BT, DI, BL), jnp.float32),
        mesh=mesh,
        scratch_types=[
            pltpu.VMEM((LT, LI, BL), jnp.int32),
            pltpu.VMEM((2, BL, D), jnp.float32),
            pltpu.VMEM((2, D, TROW), jnp.float32),
            pltpu.VMEM((4, D), jnp.float32),
            pltpu.VMEM((BL + L,), jnp.int32),
            pltpu.VMEM((BL + L,), jnp.int32),
            pltpu.SemaphoreType.DMA,
            pltpu.SemaphoreType.DMA,
            pltpu.SemaphoreType.DMA,
            pltpu.SemaphoreType.DMA,
        ],
        compiler_params=pltpu.CompilerParams(needs_layout_passes=False,
                                             use_tc_tiling_on_sc=False),
    )
    return f(idx5, prompt_params, table)


def kernel(input, table, prompt_params):
    idx = input.astype(jnp.int32)
    idx5 = idx.T.reshape(LT, LI, BT, BL).transpose(0, 2, 1, 3)
    pp = jnp.concatenate(
        [prompt_params.astype(jnp.float32),
         jnp.zeros((1, D), jnp.float32)], axis=0)
    out5 = _run(idx5, pp, table)
    return out5.transpose(2, 4, 0, 1, 3).reshape(BATCH, SEQ, D)


# speedup vs baseline: 2.3421x; 1.3818x over previous
import jax
import jax.numpy as jnp
from jax import lax
from jax.experimental import pallas as pl
from jax.experimental.pallas import tpu as pltpu
from jax.experimental.pallas import tpu_sc as plsc

VOCAB = 1000000
D = 64
PID_BASE = VOCAB - 3
NC, NS, L = 2, 16, 16
NW = NC * NS

BATCH = 4096
SEQ = 200
BL = 128
BT = BATCH // BL
LT, LI = SEQ // 8, 8
DT, DI = D // 8, 8
TROW = 129


def _sc_body(idx_hbm, pp_hbm, table_hbm, out_hbm,
             idx_v, rows_v, trans_v, pp_v, pos_v, off_v,
             sem0, sem1, osem0, osem1):
    wid = lax.axis_index("s") * NC + lax.axis_index("c")
    sems = (sem0, sem1)
    osems = (osem0, osem1)

    pltpu.sync_copy(pp_hbm, pp_v)
    for lt in range(LT):
        pltpu.sync_copy(idx_hbm.at[lt, wid], idx_v.at[lt])

    lanes = lax.iota(jnp.int32, L)

    def fire(blk, buf):
        pltpu.async_copy(table_hbm.at[idx_v.at[blk // LI, blk % LI]],
                         rows_v.at[buf], sems[buf])

    def finish(blk, buf):
        pltpu.make_async_copy(table_hbm.at[idx_v.at[blk // LI, blk % LI]],
                              rows_v.at[buf], sems[buf]).wait()

        cnt = jnp.int32(0)
        for k in range(BL // L):
            v = idx_v[blk // LI, blk % LI, pl.ds(k * L, L)]
            m = v >= PID_BASE
            loc = lanes + k * L
            plsc.store_compressed(pos_v.at[pl.ds(cnt, L)], loc, mask=m)
            plsc.store_compressed(off_v.at[pl.ds(cnt, L)], v - PID_BASE,
                                  mask=m)
            cnt = cnt + jnp.sum(m.astype(jnp.int32))

        def fix(i, c):
            p = pos_v[pl.ds(i, L)][0]
            o = off_v[pl.ds(i, L)][0]
            for kk in range(D // L):
                sl = pl.ds(kk * L, L)
                rows_v[buf, p, sl] = pp_v[o, sl]
            return c

        lax.fori_loop(0, cnt, fix, 0)

        @pl.when(blk >= 2)
        def _():
            for dt in range(DT):
                pltpu.make_async_copy(
                    trans_v.at[buf, pl.ds(dt * DI, DI), pl.ds(0, BL)],
                    out_hbm.at[blk - 2, dt, wid], osems[buf]).wait()

        @plsc.parallel_loop(0, BL, unroll=8)
        def _(r):
            cvec = jnp.broadcast_to(r, (L,)).astype(jnp.int32)
            for g4 in range(D // L):
                dvec = lanes + g4 * L
                vals = rows_v[buf, r, pl.ds(g4 * L, L)]
                plsc.store_scatter(trans_v.at[buf], [dvec, cvec], vals)

        for dt in range(DT):
            pltpu.async_copy(trans_v.at[buf, pl.ds(dt * DI, DI), pl.ds(0, BL)],
                             out_hbm.at[blk, dt, wid], osems[buf])

    fire(0, 0)

    def loop_body(t, carry):
        blk0 = 2 * t
        fire(blk0 + 1, 1)
        finish(blk0, 0)

        @pl.when(blk0 + 2 < SEQ)
        def _():
            fire(blk0 + 2, 0)

        finish(blk0 + 1, 1)
        return carry

    lax.fori_loop(0, SEQ // 2, loop_body, 0)

    for blk, buf in ((SEQ - 2, 0), (SEQ - 1, 1)):
        for dt in range(DT):
            pltpu.make_async_copy(
                trans_v.at[buf, pl.ds(dt * DI, DI), pl.ds(0, BL)],
                out_hbm.at[blk, dt, wid], osems[buf]).wait()


@jax.jit
def _run(idx5, prompt_params, table):
    mesh = plsc.VectorSubcoreMesh(core_axis_name="c", subcore_axis_name="s",
                                  num_cores=NC, num_subcores=NS)
    f = pl.kernel(
        _sc_body,
        out_type=jax.ShapeDtypeStruct((SEQ, DT, BT, DI, BL), jnp.float32),
        mesh=mesh,
        scratch_types=[
            pltpu.VMEM((LT, LI, BL), jnp.int32),
            pltpu.VMEM((2, BL, D), jnp.float32),
            pltpu.VMEM((2, D, TROW), jnp.float32),
            pltpu.VMEM((4, D), jnp.float32),
            pltpu.VMEM((BL + L,), jnp.int32),
            pltpu.VMEM((BL + L,), jnp.int32),
            pltpu.SemaphoreType.DMA,
            pltpu.SemaphoreType.DMA,
            pltpu.SemaphoreType.DMA,
            pltpu.SemaphoreType.DMA,
        ],
        compiler_params=pltpu.CompilerParams(needs_layout_passes=False,
                                             use_tc_tiling_on_sc=False),
    )
    return f(idx5, prompt_params, table)


def kernel(input, table, prompt_params):
    idx = input.astype(jnp.int32)
    idx5 = idx.T.reshape(LT, LI, BT, BL).transpose(0, 2, 1, 3)
    pp = jnp.concatenate(
        [prompt_params.astype(jnp.float32),
         jnp.zeros((1, D), jnp.float32)], axis=0)
    out5 = _run(idx5, pp, table)
    return out5.transpose(2, 4, 0, 1, 3).reshape(BATCH, SEQ, D)
